# baseline (speedup 1.0000x reference)
import functools

import jax
import jax.numpy as jnp
from jax import lax
from jax.experimental import pallas as pl
from jax.experimental.pallas import tpu as pltpu
from jax.experimental.pallas import tpu_sc as plsc

_NUM_CORES = 2
_NUM_SUBCORES = 16
_NUM_WORKERS = _NUM_CORES * _NUM_SUBCORES
_LANES = 16
_WIDE = 128


def _make_sc_lookup(n_total: int, vocab_wide: int, dim: int):
    per_w = n_total // _NUM_WORKERS
    assert per_w * _NUM_WORKERS == n_total
    chunk = 416
    assert per_w % chunk == 0
    n_chunks = per_w // chunk
    groups = chunk // _LANES

    mesh = plsc.VectorSubcoreMesh(
        core_axis_name="c", subcore_axis_name="s",
        num_cores=_NUM_CORES, num_subcores=_NUM_SUBCORES)

    @functools.partial(
        pl.kernel,
        mesh=mesh,
        compiler_params=pltpu.CompilerParams(needs_layout_passes=False),
        out_type=jax.ShapeDtypeStruct((n_total * dim,), jnp.float32),
        scratch_types=[
            pltpu.VMEM((per_w,), jnp.int32),
            pltpu.VMEM((per_w,), jnp.int32),
            pltpu.VMEM((chunk, _WIDE), jnp.float32),
            pltpu.VMEM((chunk * dim,), jnp.float32),
            pltpu.SemaphoreType.DMA,
        ],
    )
    def lookup(gidx_hbm, qoff_hbm, table_hbm, out_hbm,
               gidx_v, qoff_v, rows_v, out_v, sem):
        wid = lax.axis_index("s") * _NUM_CORES + lax.axis_index("c")
        base = wid * per_w
        pltpu.sync_copy(gidx_hbm.at[pl.ds(base, per_w)], gidx_v)
        pltpu.sync_copy(qoff_hbm.at[pl.ds(base, per_w)], qoff_v)
        iota = lax.iota(jnp.int32, _LANES)

        def chunk_body(ci, carry):
            pltpu.async_copy(
                table_hbm.at[gidx_v.at[pl.ds(ci * chunk, chunk)]],
                rows_v, sem).wait()

            def group_body(g, gcarry):
                rvec = g * _LANES + iota
                qv = qoff_v[pl.ds(ci * chunk + g * _LANES, _LANES)]
                ob = rvec * dim
                for c in range(dim):
                    x = plsc.load_gather(rows_v, [rvec, qv + c])
                    s = 1.0 / (1.0 + jnp.exp(-x))
                    plsc.store_scatter(out_v, [ob + c], s)
                return gcarry

            lax.fori_loop(0, groups, group_body, 0)
            pltpu.sync_copy(
                out_v,
                out_hbm.at[pl.ds((base + ci * chunk) * dim, chunk * dim)])
            return carry

        lax.fori_loop(0, n_chunks, chunk_body, 0)

    return lookup


def kernel(features, embedding_table):
    b, f = features.shape
    v, d = embedding_table.shape
    n = b * f
    rpg = _WIDE // d
    idx = features.reshape(n)
    gidx = idx // rpg
    qoff = (idx % rpg) * d
    table_wide = embedding_table.reshape(v // rpg, _WIDE)
    lookup = _make_sc_lookup(n, v // rpg, d)
    out = lookup(gidx, qoff, table_wide)
    return out.reshape(b, f, d)


# speedup vs baseline: 1.5537x; 1.5537x over previous
import functools

import jax
import jax.numpy as jnp
from jax import lax
from jax.experimental import pallas as pl
from jax.experimental.pallas import tpu as pltpu
from jax.experimental.pallas import tpu_sc as plsc

_NUM_CORES = 2
_NUM_SUBCORES = 16
_NUM_WORKERS = _NUM_CORES * _NUM_SUBCORES
_LANES = 16
_WIDE = 128


def _make_sc_lookup(batch: int, fields: int, vocab_wide: int, dim: int):
    bw = batch // _NUM_WORKERS
    assert bw * _NUM_WORKERS == batch
    groups = bw // _LANES

    mesh = plsc.VectorSubcoreMesh(
        core_axis_name="c", subcore_axis_name="s",
        num_cores=_NUM_CORES, num_subcores=_NUM_SUBCORES)

    @functools.partial(
        pl.kernel,
        mesh=mesh,
        compiler_params=pltpu.CompilerParams(needs_layout_passes=False),
        out_type=jax.ShapeDtypeStruct((fields, dim, batch), jnp.float32),
        scratch_types=[
            pltpu.VMEM((bw,), jnp.int32),
            pltpu.VMEM((bw,), jnp.int32),
            pltpu.VMEM((bw, _WIDE), jnp.float32),
            pltpu.VMEM((dim, bw), jnp.float32),
            pltpu.SemaphoreType.DMA,
        ],
    )
    def lookup(gidx_hbm, qoff_hbm, table_hbm, out_hbm,
               gidx_v, qoff_v, rows_v, out_t, sem):
        wid = lax.axis_index("s") * _NUM_CORES + lax.axis_index("c")
        b0 = wid * bw
        iota = lax.iota(jnp.int32, _LANES)

        def field_body(f, carry):
            base = f * batch + b0
            pltpu.sync_copy(gidx_hbm.at[pl.ds(base, bw)], gidx_v)
            pltpu.sync_copy(qoff_hbm.at[pl.ds(base, bw)], qoff_v)
            pltpu.async_copy(table_hbm.at[gidx_v], rows_v, sem).wait()

            def group_body(g, gcarry):
                rvec = g * _LANES + iota
                qv = qoff_v[pl.ds(g * _LANES, _LANES)]
                for c in range(dim):
                    cv = (iota + c) & (dim - 1)
                    x = plsc.load_gather(rows_v, [rvec, qv + cv])
                    s = 1.0 / (1.0 + jnp.exp(-x))
                    plsc.store_scatter(out_t, [cv, rvec], s)
                return gcarry

            lax.fori_loop(0, groups, group_body, 0)
            pltpu.sync_copy(out_t, out_hbm.at[f, :, pl.ds(b0, bw)])
            return carry

        lax.fori_loop(0, fields, field_body, 0)

    return lookup


def kernel(features, embedding_table):
    b, f = features.shape
    v, d = embedding_table.shape
    rpg = _WIDE // d
    idx = features.T.reshape(f * b)
    gidx = lax.shift_right_logical(idx, 2)
    qoff = (idx & (rpg - 1)) * d
    table_wide = embedding_table.reshape(v // rpg, _WIDE)
    lookup = _make_sc_lookup(b, f, v // rpg, d)
    out = lookup(gidx, qoff, table_wide)
    return out.transpose(2, 0, 1)


# speedup vs baseline: 1.8656x; 1.2007x over previous
import functools

import jax
import jax.numpy as jnp
from jax import lax
from jax.experimental import pallas as pl
from jax.experimental.pallas import tpu as pltpu
from jax.experimental.pallas import tpu_sc as plsc

_NUM_CORES = 2
_NUM_SUBCORES = 16
_NUM_WORKERS = _NUM_CORES * _NUM_SUBCORES
_LANES = 16
_WIDE = 128


def _make_sc_lookup(batch: int, fields: int, vocab_wide: int, dim: int):
    bw = batch // _NUM_WORKERS
    assert bw * _NUM_WORKERS == batch
    groups = bw // _LANES

    mesh = plsc.VectorSubcoreMesh(
        core_axis_name="c", subcore_axis_name="s",
        num_cores=_NUM_CORES, num_subcores=_NUM_SUBCORES)

    @functools.partial(
        pl.kernel,
        mesh=mesh,
        compiler_params=pltpu.CompilerParams(needs_layout_passes=False),
        out_type=jax.ShapeDtypeStruct((fields, dim, batch), jnp.float32),
        scratch_types=[
            pltpu.VMEM((bw,), jnp.int32),
            pltpu.VMEM((bw,), jnp.int32),
            pltpu.VMEM((bw, _WIDE), jnp.float32),
            pltpu.VMEM((dim, bw), jnp.float32),
            pltpu.SemaphoreType.DMA,
        ],
    )
    def lookup(gidx_hbm, qoff_hbm, table_hbm, out_hbm,
               gidx_v, qoff_v, rows_v, out_t, sem):
        wid = lax.axis_index("s") * _NUM_CORES + lax.axis_index("c")
        b0 = wid * bw
        iota = lax.iota(jnp.int32, _LANES)

        def field_body(f, carry):
            base = f * batch + b0
            pltpu.sync_copy(gidx_hbm.at[pl.ds(base, bw)], gidx_v)
            pltpu.sync_copy(qoff_hbm.at[pl.ds(base, bw)], qoff_v)
            pltpu.async_copy(table_hbm.at[gidx_v], rows_v, sem).wait()

            @plsc.parallel_loop(0, groups, 1, unroll=2)
            def group_body(g):
                rvec = g * _LANES + iota
                qv = qoff_v[pl.ds(g * _LANES, _LANES)]
                for c in range(dim):
                    cv = (iota + c) & (dim - 1)
                    x = plsc.load_gather(rows_v, [rvec, qv + cv])
                    s = 1.0 / (1.0 + jnp.exp(-x))
                    plsc.store_scatter(out_t, [cv, rvec], s)
            pltpu.sync_copy(out_t, out_hbm.at[f, :, pl.ds(b0, bw)])
            return carry

        lax.fori_loop(0, fields, field_body, 0)

    return lookup


def kernel(features, embedding_table):
    b, f = features.shape
    v, d = embedding_table.shape
    rpg = _WIDE // d
    idx = features.T.reshape(f * b)
    gidx = lax.shift_right_logical(idx, 2)
    qoff = (idx & (rpg - 1)) * d
    table_wide = embedding_table.reshape(v // rpg, _WIDE)
    lookup = _make_sc_lookup(b, f, v // rpg, d)
    out = lookup(gidx, qoff, table_wide)
    return out.transpose(2, 0, 1)


# speedup vs baseline: 2.0280x; 1.0870x over previous
import functools

import jax
import jax.numpy as jnp
from jax import lax
from jax.experimental import pallas as pl
from jax.experimental.pallas import tpu as pltpu
from jax.experimental.pallas import tpu_sc as plsc

_NUM_CORES = 2
_NUM_SUBCORES = 16
_NUM_WORKERS = _NUM_CORES * _NUM_SUBCORES
_LANES = 16
_WIDE = 128


def _make_sc_lookup(batch: int, fields: int, vocab_wide: int, dim: int):
    bw = batch // _NUM_WORKERS
    assert bw * _NUM_WORKERS == batch
    groups = bw // _LANES

    mesh = plsc.VectorSubcoreMesh(
        core_axis_name="c", subcore_axis_name="s",
        num_cores=_NUM_CORES, num_subcores=_NUM_SUBCORES)

    @functools.partial(
        pl.kernel,
        mesh=mesh,
        compiler_params=pltpu.CompilerParams(needs_layout_passes=False),
        out_type=jax.ShapeDtypeStruct((fields, dim, batch), jnp.float32),
        scratch_types=[
            pltpu.VMEM((bw,), jnp.int32),
            pltpu.VMEM((bw,), jnp.int32),
            pltpu.VMEM((bw, _WIDE), jnp.float32),
            pltpu.VMEM((dim, bw), jnp.float32),
            pltpu.SemaphoreType.DMA,
        ],
    )
    def lookup(gidx_hbm, qoff_hbm, table_hbm, out_hbm,
               gidx_v, qoff_v, rows_v, out_t, sem):
        wid = lax.axis_index("s") * _NUM_CORES + lax.axis_index("c")
        b0 = wid * bw
        iota = lax.iota(jnp.int32, _LANES)

        def field_body(f, carry):
            base = f * batch + b0
            pltpu.sync_copy(gidx_hbm.at[pl.ds(base, bw)], gidx_v)
            pltpu.sync_copy(qoff_hbm.at[pl.ds(base, bw)], qoff_v)
            pltpu.async_copy(table_hbm.at[gidx_v], rows_v, sem).wait()

            @plsc.parallel_loop(0, groups, 1, unroll=2)
            def group_body(g):
                rvec = g * _LANES + iota
                qv = qoff_v[pl.ds(g * _LANES, _LANES)]
                for c in range(dim):
                    cv = (iota + c) & (dim - 1)
                    x = plsc.load_gather(rows_v, [rvec, qv + cv])
                    s = 1.0 / (1.0 + jnp.exp(-x))
                    plsc.store_scatter(out_t, [cv, rvec], s)
            pltpu.sync_copy(out_t, out_hbm.at[f, :, pl.ds(b0, bw)])
            return carry

        lax.fori_loop(0, fields, field_body, 0)

    return lookup


_VCOLS = 2048
_SUB = _VCOLS // 4


def _tc_widen(table_t, vocab: int, dim: int):
    rpg = _WIDE // dim
    out_rows = _VCOLS // rpg
    grid = pl.cdiv(vocab, _VCOLS)

    def tbody(x_ref, o_ref):
        x = x_ref[...]
        o_ref[...] = jnp.concatenate(
            [x[:, s * _SUB:(s + 1) * _SUB].T for s in range(rpg)], axis=1)

    return pl.pallas_call(
        tbody,
        grid=(grid,),
        in_specs=[pl.BlockSpec((dim, _VCOLS), lambda i: (0, i))],
        out_specs=pl.BlockSpec((out_rows, _WIDE), lambda i: (i, 0)),
        out_shape=jax.ShapeDtypeStruct((grid * out_rows, _WIDE), jnp.float32),
    )(table_t)


def kernel(features, embedding_table):
    b, f = features.shape
    v, d = embedding_table.shape
    idx = features.T.reshape(f * b)
    gidx = ((idx >> 11) << 9) + (idx & (_SUB - 1))
    qoff = ((idx >> 9) & 3) * d
    table_wide = _tc_widen(embedding_table.T, v, d)
    lookup = _make_sc_lookup(b, f, table_wide.shape[0], d)
    out = lookup(gidx, qoff, table_wide)
    return out.transpose(2, 0, 1)
